# baseline (speedup 1.0000x reference)
import functools

import jax
import jax.numpy as jnp
from jax import lax
from jax.experimental import pallas as pl
from jax.experimental.pallas import tpu as pltpu
from jax.experimental.pallas import tpu_sc as plsc

NUM_IDS = 1000000
FEAT_DIM = 64
OUT_DIM = 128
BATCH = 16384

_NC = 2
_NS = 16
_NW = _NC * _NS
_BPW = BATCH // _NW
_CHUNK = 128
_NCHUNK = _BPW // _CHUNK


def _sc_gather(ids2d, table):
    mesh = plsc.VectorSubcoreMesh(core_axis_name="c", subcore_axis_name="s")

    @functools.partial(
        pl.kernel,
        mesh=mesh,
        out_type=jax.ShapeDtypeStruct((BATCH, FEAT_DIM), jnp.float32),
        scratch_types=[
            pltpu.VMEM((_NCHUNK, _CHUNK), jnp.int32),
            pltpu.VMEM((_BPW, FEAT_DIM), jnp.float32),
            pltpu.SemaphoreType.DMA,
        ],
        compiler_params=pltpu.CompilerParams(use_tc_tiling_on_sc=False),
    )
    def gather_kernel(ids_hbm, table_hbm, out_hbm, idx_v, rows_v, sem):
        wid = lax.axis_index("s") * _NC + lax.axis_index("c")
        pltpu.sync_copy(ids_hbm.at[pl.ds(wid * _NCHUNK, _NCHUNK)], idx_v)
        copies = [
            pltpu.async_copy(
                table_hbm.at[idx_v.at[j]],
                rows_v.at[pl.ds(j * _CHUNK, _CHUNK)],
                sem,
            )
            for j in range(_NCHUNK)
        ]
        for cp in copies:
            cp.wait()
        pltpu.sync_copy(rows_v, out_hbm.at[pl.ds(wid * _BPW, _BPW)])

    return gather_kernel(ids2d, table)


def _ln(x):
    mu = jnp.mean(x, axis=-1, keepdims=True)
    var = jnp.mean((x - mu) * (x - mu), axis=-1, keepdims=True)
    return (x - mu) * lax.rsqrt(var + 1e-5)


def _gelu(x):
    return x * 0.5 * (1.0 + lax.erf(x * 0.7071067811865476))


_BB = 1024
_NB = BATCH // _BB
_HI = lax.Precision.HIGHEST


def _mlp_body(idemb_ref, g_ref, a_ref, o_ref, z_ref, w1a_ref, w1b_ref,
              b1_ref, w2_ref, b2_ref, w3_ref, b3_ref, out_ref):
    g = g_ref[0]
    a = a_ref[0]
    o = o_ref[0]
    iota = lax.broadcasted_iota(jnp.int32, (32, _BB), 0)
    tgt = jnp.where(iota < 2, g, jnp.where(iota < 9, a + 2, o + 9))
    mh = (iota == tgt).astype(jnp.float32)
    p = lax.dot_general(z_ref[...], w1b_ref[...], (((1,), (0,)), ((), ())),
                        precision=_HI)
    hc = lax.dot_general(mh, p, (((0,), (0,)), ((), ())), precision=_HI)
    h = lax.dot_general(idemb_ref[...], w1a_ref[...], (((1,), (0,)), ((), ())),
                        precision=_HI) + hc + b1_ref[...]
    h = _gelu(_ln(h))
    h = lax.dot_general(h, w2_ref[...], (((1,), (0,)), ((), ())),
                        precision=_HI) + b2_ref[...]
    h = _gelu(_ln(h))
    h = lax.dot_general(h, w3_ref[...], (((1,), (0,)), ((), ())),
                        precision=_HI) + b3_ref[...]
    out_ref[...] = _gelu(h)


def _mlp(id_emb, g3, a3, o3, z, w1a, w1b, b1r, w2, b2r, w3, b3r,
         interpret=False):
    full = lambda shape: pl.BlockSpec(shape, lambda i: (0,) * len(shape))
    return pl.pallas_call(
        _mlp_body,
        grid=(_NB,),
        in_specs=[
            pl.BlockSpec((_BB, FEAT_DIM), lambda i: (i, 0)),
            pl.BlockSpec((1, 1, _BB), lambda i: (i, 0, 0)),
            pl.BlockSpec((1, 1, _BB), lambda i: (i, 0, 0)),
            pl.BlockSpec((1, 1, _BB), lambda i: (i, 0, 0)),
            full((32, 3 * FEAT_DIM)),
            full((FEAT_DIM, 128)),
            full((3 * FEAT_DIM, 128)),
            full((1, 128)),
            full((128, 64)),
            full((1, 64)),
            full((64, OUT_DIM)),
            full((1, OUT_DIM)),
        ],
        out_specs=pl.BlockSpec((_BB, OUT_DIM), lambda i: (i, 0)),
        out_shape=jax.ShapeDtypeStruct((BATCH, OUT_DIM), jnp.float32),
        interpret=interpret,
    )(id_emb, g3, a3, o3, z, w1a, w1b, b1r, w2, b2r, w3, b3r)


def kernel(user_ids, user_genders, user_ages, user_occs, id_table,
           gender_table, age_table, occ_table, W1, b1, W2, b2, W3, b3):
    ids2d = user_ids.reshape(_NW * _NCHUNK, _CHUNK)
    id_emb = _sc_gather(ids2d, id_table)

    z = jnp.zeros((32, 3 * FEAT_DIM), jnp.float32)
    z = z.at[0:2, 0:FEAT_DIM].set(gender_table)
    z = z.at[2:9, FEAT_DIM:2 * FEAT_DIM].set(age_table)
    z = z.at[9:30, 2 * FEAT_DIM:3 * FEAT_DIM].set(occ_table)

    g3 = user_genders.reshape(_NB, 1, _BB)
    a3 = user_ages.reshape(_NB, 1, _BB)
    o3 = user_occs.reshape(_NB, 1, _BB)

    return _mlp(id_emb, g3, a3, o3, z,
                W1[:FEAT_DIM], W1[FEAT_DIM:],
                b1.reshape(1, -1), W2, b2.reshape(1, -1),
                W3, b3.reshape(1, -1))
